# Initial kernel scaffold; baseline (speedup 1.0000x reference)
import jax
import jax.numpy as jnp
from jax.experimental import pallas as pl


def kernel(hidden_states, Wg, Wn, W_down, W_up):
    raise NotImplementedError("write your pallas kernel here")


# speedup vs baseline: 2.6896x; 2.6896x over previous
import functools

import jax
import jax.numpy as jnp
from jax.experimental import pallas as pl
from jax.experimental.pallas import tpu as pltpu

NUM_EXPERTS = 8
TOP_K = 2
RANK = 128
D_IN = 2048
D_OUT = 2048
BLK = 512


def _moe_body(x_ref, wg_ref, wn_ref, wd_ref, wu_ref, noise_ref,
              out_ref, rl_ref):
    x = x_ref[...]

    logits = jax.lax.dot_general(
        x, wg_ref[...], (((1,), (1,)), ((), ())),
        preferred_element_type=jnp.float32)
    nlogits = jax.lax.dot_general(
        x, wn_ref[...], (((1,), (1,)), ((), ())),
        preferred_element_type=jnp.float32)
    rl = logits + noise_ref[...] * jax.nn.softplus(nlogits)
    rl_ref[...] = rl

    p = jax.nn.softmax(rl, axis=-1)

    col = jax.lax.broadcasted_iota(jnp.int32, p.shape, 1)
    m1 = jnp.max(p, axis=-1, keepdims=True)
    a1 = jnp.min(jnp.where(p == m1, col, NUM_EXPERTS), axis=-1, keepdims=True)
    first = col == a1
    p_m = jnp.where(first, -jnp.inf, p)
    m2 = jnp.max(p_m, axis=-1, keepdims=True)
    a2 = jnp.min(jnp.where(p_m == m2, col, NUM_EXPERTS), axis=-1, keepdims=True)
    sel = first | (col == a2)
    w = jnp.where(sel, p, 0.0)
    w = w / jnp.sum(w, axis=-1, keepdims=True)

    acc = jnp.zeros((x.shape[0], D_OUT), jnp.float32)
    for e in range(NUM_EXPERTS):
        down = jax.lax.dot_general(
            x, wd_ref[e], (((1,), (1,)), ((), ())),
            preferred_element_type=jnp.float32)
        up = jax.lax.dot_general(
            down, wu_ref[e], (((1,), (1,)), ((), ())),
            preferred_element_type=jnp.float32)
        acc = acc + up * w[:, e:e + 1]
    out_ref[...] = acc


@functools.partial(jax.jit, static_argnames=("interpret",))
def kernel(hidden_states, Wg, Wn, W_down, W_up, interpret=False):
    b, s, d = hidden_states.shape
    T = b * s
    x = hidden_states.reshape(T, d)
    noise = jax.random.normal(jax.random.key(42), (T, NUM_EXPERTS),
                              jnp.float32)

    grid = (T // BLK,)
    out, rl = pl.pallas_call(
        _moe_body,
        grid=grid,
        in_specs=[
            pl.BlockSpec((BLK, D_IN), lambda i: (i, 0)),
            pl.BlockSpec((NUM_EXPERTS, D_IN), lambda i: (0, 0)),
            pl.BlockSpec((NUM_EXPERTS, D_IN), lambda i: (0, 0)),
            pl.BlockSpec((NUM_EXPERTS, RANK, D_IN), lambda i: (0, 0, 0)),
            pl.BlockSpec((NUM_EXPERTS, D_OUT, RANK), lambda i: (0, 0, 0)),
            pl.BlockSpec((BLK, NUM_EXPERTS), lambda i: (i, 0)),
        ],
        out_specs=[
            pl.BlockSpec((BLK, D_OUT), lambda i: (i, 0)),
            pl.BlockSpec((BLK, NUM_EXPERTS), lambda i: (i, 0)),
        ],
        out_shape=[
            jax.ShapeDtypeStruct((T, D_OUT), jnp.float32),
            jax.ShapeDtypeStruct((T, NUM_EXPERTS), jnp.float32),
        ],
        compiler_params=pltpu.CompilerParams(
            dimension_semantics=("arbitrary",),
        ),
        interpret=interpret,
    )(x, Wg, Wn, W_down, W_up, noise)
    return out.reshape(b, s, D_OUT), rl
